# baseline (speedup 1.0000x reference)
import jax
import jax.numpy as jnp
from jax import lax
from jax.experimental import pallas as pl
from jax.experimental.pallas import tpu as pltpu
from jax.experimental.pallas import tpu_sc as plsc

NUM_USERS = 100000
D = 64
B = 16384
NC = 2
NS = 16
NW = NC * NS
BPW = B // NW
CH = 128
NCH = BPW // CH


def _sc_dot_kernel(src_hbm, tgt_hbm, table_hbm, out_hbm,
                   sidx, tidx, rows_s, rows_t, out_v, sem):
    wid = lax.axis_index("s") * NC + lax.axis_index("c")

    pltpu.sync_copy(src_hbm.at[wid], sidx)
    pltpu.sync_copy(tgt_hbm.at[wid], tidx)

    copies = []
    for j in range(NCH):
        copies.append(pltpu.async_copy(
            table_hbm.at[sidx.at[j]], rows_s.at[pl.ds(j * CH, CH)], sem))
        copies.append(pltpu.async_copy(
            table_hbm.at[tidx.at[j]], rows_t.at[pl.ds(j * CH, CH)], sem))
    for c in copies:
        c.wait()

    lanes = lax.iota(jnp.int32, 16)

    def block(rb, carry):
        ridx = rb * 16 + lanes
        acc = jnp.zeros((16,), jnp.float32)

        def dstep(d, acc):
            col = jnp.full((16,), d, dtype=jnp.int32)
            a = plsc.load_gather(rows_s, [ridx, col])
            b = plsc.load_gather(rows_t, [ridx, col])
            return acc + a * b

        acc = lax.fori_loop(0, D, dstep, acc, unroll=8)
        out_v[pl.ds(rb * 16, 16)] = acc
        return carry

    lax.fori_loop(0, BPW // 16, block, 0)

    pltpu.sync_copy(out_v, out_hbm.at[pl.ds(wid * BPW, BPW)])


def _make_call():
    mesh = plsc.VectorSubcoreMesh(core_axis_name="c", subcore_axis_name="s",
                                  num_cores=NC, num_subcores=NS)
    return pl.kernel(
        _sc_dot_kernel,
        out_type=jax.ShapeDtypeStruct((B,), jnp.float32),
        mesh=mesh,
        compiler_params=pltpu.CompilerParams(
            use_tc_tiling_on_sc=False, needs_layout_passes=False),
        scratch_types=[
            pltpu.VMEM((NCH, CH), jnp.int32),
            pltpu.VMEM((NCH, CH), jnp.int32),
            pltpu.VMEM((BPW, D), jnp.float32),
            pltpu.VMEM((BPW, D), jnp.float32),
            pltpu.VMEM((BPW,), jnp.float32),
            pltpu.SemaphoreType.DMA,
        ],
    )


_call = _make_call()


@jax.jit
def kernel(src, tgt, embedding_user):
    src_r = src.astype(jnp.int32).reshape(NW, NCH, CH)
    tgt_r = tgt.astype(jnp.int32).reshape(NW, NCH, CH)
    return _call(src_r, tgt_r, embedding_user)


# speedup vs baseline: 1.4781x; 1.4781x over previous
import jax
import jax.numpy as jnp
from jax import lax
from jax.experimental import pallas as pl
from jax.experimental.pallas import tpu as pltpu
from jax.experimental.pallas import tpu_sc as plsc

NUM_USERS = 100000
D = 64
B = 16384
NC = 2
NS = 16
NW = NC * NS
BPW = B // NW
CH = 128
NCH = BPW // CH

UB = 1024
HALF = 50176
GRID = HALF // UB


def _reformat_kernel(lo_ref, hi_ref, out_ref):
    out_ref[:, 0:D] = lo_ref[...].T
    out_ref[:, D:2 * D] = hi_ref[...].T


_reformat = pl.pallas_call(
    _reformat_kernel,
    grid=(GRID,),
    in_specs=[
        pl.BlockSpec((D, UB), lambda i: (0, i)),
        pl.BlockSpec((D, UB), lambda i: (0, i + GRID)),
    ],
    out_specs=pl.BlockSpec((UB, 2 * D), lambda i: (i, 0)),
    out_shape=jax.ShapeDtypeStruct((HALF, 2 * D), jnp.float32),
)


def _sc_dot_kernel(src_hbm, tgt_hbm, table_hbm, out_hbm,
                   sidx, tidx, srow, trow, soff, toff,
                   bs, bt, out_v, sem0, sem1):
    wid = lax.axis_index("s") * NC + lax.axis_index("c")
    base = wid * BPW

    pltpu.sync_copy(src_hbm.at[pl.ds(base, BPW)], sidx)
    pltpu.sync_copy(tgt_hbm.at[pl.ds(base, BPW)], tidx)

    for c in range(BPW // 16):
        sl = pl.ds(c * 16, 16)
        for idx, row, off in ((sidx, srow, soff), (tidx, trow, toff)):
            v = idx[sl]
            hi = v >= HALF
            row[sl] = jnp.where(hi, v - HALF, v)
            off[sl] = jnp.where(hi, D, 0)

    sems = (sem0, sem1)

    def start_gather(j):
        slot = j % 2
        jsl = pl.ds(j * CH, CH)
        return (
            pltpu.async_copy(table_hbm.at[srow.at[jsl]], bs.at[slot],
                             sems[slot]),
            pltpu.async_copy(table_hbm.at[trow.at[jsl]], bt.at[slot],
                             sems[slot]),
        )

    lanes = lax.iota(jnp.int32, 16)
    masks = [lanes == r16 for r16 in range(16)]

    def compute_chunk(j, slot):
        rs = bs.at[slot]
        rt = bt.at[slot]

        def blk_body(blk, carry):
            ob = jnp.zeros((16,), jnp.float32)
            ovs = soff[pl.ds(j * CH + blk * 16, 16)]
            ovt = toff[pl.ds(j * CH + blk * 16, 16)]
            for r16 in range(16):
                r = blk * 16 + r16
                os = ovs[r16]
                ot = ovt[r16]
                p = rs[r, pl.ds(os, 16)] * rt[r, pl.ds(ot, 16)]
                for k in range(1, D // 16):
                    p += (rs[r, pl.ds(os + k * 16, 16)]
                          * rt[r, pl.ds(ot + k * 16, 16)])
                ob = jnp.where(masks[r16], jnp.sum(p), ob)
            out_v[pl.ds(j * CH + blk * 16, 16)] = ob
            return carry

        lax.fori_loop(0, CH // 16, blk_body, 0)

    pending = start_gather(0)
    for j in range(NCH):
        nxt = start_gather(j + 1) if j + 1 < NCH else None
        for c in pending:
            c.wait()
        compute_chunk(j, j % 2)
        pending = nxt

    pltpu.sync_copy(out_v, out_hbm.at[pl.ds(base, BPW)])


def _make_sc_call():
    mesh = plsc.VectorSubcoreMesh(core_axis_name="c", subcore_axis_name="s",
                                  num_cores=NC, num_subcores=NS)
    return pl.kernel(
        _sc_dot_kernel,
        out_type=jax.ShapeDtypeStruct((B,), jnp.float32),
        mesh=mesh,
        compiler_params=pltpu.CompilerParams(needs_layout_passes=False),
        scratch_types=[
            pltpu.VMEM((BPW,), jnp.int32),
            pltpu.VMEM((BPW,), jnp.int32),
            pltpu.VMEM((BPW,), jnp.int32),
            pltpu.VMEM((BPW,), jnp.int32),
            pltpu.VMEM((BPW,), jnp.int32),
            pltpu.VMEM((BPW,), jnp.int32),
            pltpu.VMEM((2, CH, 2 * D), jnp.float32),
            pltpu.VMEM((2, CH, 2 * D), jnp.float32),
            pltpu.VMEM((BPW,), jnp.float32),
            pltpu.SemaphoreType.DMA,
            pltpu.SemaphoreType.DMA,
        ],
    )


_sc_call = _make_sc_call()


@jax.jit
def kernel(src, tgt, embedding_user):
    table2 = _reformat(embedding_user.T, embedding_user.T)
    return _sc_call(src.astype(jnp.int32), tgt.astype(jnp.int32), table2)
